# baseline (speedup 1.0000x reference)
import functools

import jax
import jax.numpy as jnp
from jax import lax
from jax.experimental import pallas as pl
from jax.experimental.pallas import tpu as pltpu
from jax.experimental.pallas import tpu_sc as plsc

N = 50000
E = 800000
V = 10000
F = 64
FH = 32
CLS = 10
G = 512

NC = 2
NS = 16
NW = NC * NS

N_PAD = 51200
ROWS_PER_TILE = N_PAD // NS
C = 128
NCHUNK = E // C
EMB_CHUNKS = ROWS_PER_TILE // C
NBUF = 4


def _mesh():
    return plsc.VectorSubcoreMesh(
        core_axis_name="c", subcore_axis_name="s", num_cores=NC, num_subcores=NS)


_SC_PARAMS = pltpu.CompilerParams(use_tc_tiling_on_sc=False)


def _b(j):
    return jnp.bitwise_and(j, NBUF - 1)


def _edge_phase(edge_ref, h_ref, acc_sh, ebuf, rows, semi, semg, semr,
                c, s, ones=None, cnt_sh=None, semo=None):
    lo = s * NCHUNK // NS
    hi = (s + 1) * NCHUNK // NS
    with_ones = ones is not None

    def idx_copy(j):
        return pltpu.make_async_copy(
            edge_ref.at[:, pl.ds(j * C, C)], ebuf.at[_b(j)], semi.at[_b(j)])

    def gather_copy(j):
        return pltpu.make_async_copy(
            h_ref.at[ebuf.at[_b(j), c]], rows.at[_b(j)], semg.at[_b(j)])

    def scat_start(j):
        pltpu.async_copy(
            rows.at[_b(j)], acc_sh.at[ebuf.at[_b(j), 2]], semr.at[_b(j)],
            add=True)

    def scat_wait(j):
        pltpu.make_async_copy(
            rows.at[_b(j)], acc_sh.at[ebuf.at[_b(j), 2]], semr.at[_b(j)]).wait()

    def ones_start(j):
        pltpu.async_copy(
            ones, cnt_sh.at[ebuf.at[_b(j), 2]], semo.at[_b(j)], add=True)

    def ones_wait(j):
        pltpu.make_async_copy(
            ones, cnt_sh.at[ebuf.at[_b(j), 2]], semo.at[_b(j)]).wait()

    @pl.when(lo < hi)
    def _():
        idx_copy(lo).start()

    @pl.when(lo + 1 < hi)
    def _():
        idx_copy(lo + 1).start()

    @pl.when(lo < hi)
    def _():
        idx_copy(lo).wait()
        gather_copy(lo).start()

    def body(j, _):
        @pl.when(j - 2 >= lo)
        def _():
            scat_wait(j - 2)
            if with_ones:
                @pl.when(c == 0)
                def _():
                    ones_wait(j - 2)

        @pl.when(j + 2 < hi)
        def _():
            idx_copy(j + 2).start()

        @pl.when(j + 1 < hi)
        def _():
            idx_copy(j + 1).wait()
            gather_copy(j + 1).start()

        gather_copy(j).wait()
        scat_start(j)
        if with_ones:
            @pl.when(c == 0)
            def _():
                ones_start(j)
        return 0
    lax.fori_loop(lo, hi, body, 0)

    def drain(j, _):
        scat_wait(j)
        if with_ones:
            @pl.when(c == 0)
            def _():
                ones_wait(j)
        return 0
    lax.fori_loop(jnp.maximum(lo, hi - 2), hi, drain, 0)


def _zero_rows(zrows):
    def zloop(r, _):
        zrows[r, pl.ds(0, 16)] = jnp.zeros((16,), jnp.float32)
        zrows[r, pl.ds(16, 16)] = jnp.zeros((16,), jnp.float32)
        return 0
    lax.fori_loop(0, C, zloop, 0)


def _acc_writeout(acc_sh, out_ref, rows, c, s):
    for t in range(EMB_CHUNKS):
        base = s * ROWS_PER_TILE + t * C
        pltpu.sync_copy(acc_sh.at[pl.ds(base, C)], rows.at[0])
        pltpu.sync_copy(rows.at[0], out_ref.at[pl.ds(c * N_PAD + base, C)])


def _stage_ab1(x2, edge3, emb_cat):
    @functools.partial(
        pl.kernel,
        out_type=(
            jax.ShapeDtypeStruct((2 * N_PAD, FH), jnp.float32),
            jax.ShapeDtypeStruct((2 * N_PAD, FH), jnp.float32),
            jax.ShapeDtypeStruct((N_PAD,), jnp.float32),
        ),
        mesh=_mesh(),
        scratch_types=[
            pltpu.VMEM((NBUF, 2, C), jnp.int32),
            pltpu.VMEM((NBUF, 3, C), jnp.int32),
            pltpu.VMEM((NBUF, C, FH), jnp.float32),
            pltpu.VMEM((C, FH), jnp.float32),
            pltpu.VMEM((C,), jnp.float32),
            pltpu.VMEM((800,), jnp.float32),
            pltpu.VMEM((800,), jnp.float32),
            pltpu.VMEM_SHARED((N_PAD, FH), jnp.float32),
            pltpu.VMEM_SHARED((N_PAD,), jnp.float32),
            pltpu.SemaphoreType.DMA((NBUF,)),
            pltpu.SemaphoreType.DMA((NBUF,)),
            pltpu.SemaphoreType.DMA((NBUF,)),
            pltpu.SemaphoreType.DMA((NBUF,)),
            pltpu.SemaphoreType.DMA((NBUF,)),
        ],
        compiler_params=_SC_PARAMS,
    )
    def run(x_ref, edge_ref, emb_ref, h0_ref, agg_ref, cnt_ref,
            xbuf, ebuf, rows, zrows, ones, zbuf, sbuf, acc_sh, cnt_sh,
            semx, semi, semg, semr, semo):
        c = lax.axis_index("c")
        s = lax.axis_index("s")

        _zero_rows(zrows)

        def oloop(i, _):
            ones[pl.ds(i * 16, 16)] = jnp.ones((16,), jnp.float32)
            return 0
        lax.fori_loop(0, C // 16, oloop, 0)

        def zvloop(i, _):
            zbuf[pl.ds(i * 16, 16)] = jnp.zeros((16,), jnp.float32)
            return 0
        lax.fori_loop(0, 50, zvloop, 0)

        for t in range(EMB_CHUNKS):
            pltpu.sync_copy(zrows, acc_sh.at[pl.ds(s * ROWS_PER_TILE + t * C, C)])
        for t in range(4):
            pltpu.sync_copy(zbuf, cnt_sh.at[pl.ds(s * ROWS_PER_TILE + t * 800, 800)])

        def xload(k):
            base = s * ROWS_PER_TILE + k * C
            return pltpu.make_async_copy(
                x_ref.at[:, pl.ds(base, C)], xbuf.at[_b(k)], semx.at[_b(k)])

        def embg(k):
            return pltpu.make_async_copy(
                emb_ref.at[xbuf.at[_b(k), c]], rows.at[_b(k)], semg.at[_b(k)])

        xload(0).start()
        xload(1).start()
        xload(0).wait()
        embg(0).start()

        def emb_body(k, _):
            @pl.when(k + 2 < EMB_CHUNKS)
            def _():
                xload(k + 2).start()

            @pl.when(k + 1 < EMB_CHUNKS)
            def _():
                xload(k + 1).wait()
                embg(k + 1).start()

            embg(k).wait()
            base = s * ROWS_PER_TILE + k * C
            pltpu.sync_copy(rows.at[_b(k)],
                            h0_ref.at[pl.ds(c * N_PAD + base, C)])
            return 0
        lax.fori_loop(0, EMB_CHUNKS, emb_body, 0)

        plsc.subcore_barrier()

        _edge_phase(edge_ref, h0_ref, acc_sh, ebuf, rows, semi, semg, semr,
                    c, s, ones=ones, cnt_sh=cnt_sh, semo=semo)

        plsc.subcore_barrier()

        _acc_writeout(acc_sh, agg_ref, rows, c, s)

        @pl.when(c == 0)
        def _():
            for t in range(4):
                sl = pl.ds(s * ROWS_PER_TILE + t * 800, 800)
                pltpu.sync_copy(cnt_sh.at[sl], sbuf)
                pltpu.sync_copy(sbuf, cnt_ref.at[sl])

    return run(x2, edge3, emb_cat)


def _stage_b(edge3, h_cat):
    @functools.partial(
        pl.kernel,
        out_type=jax.ShapeDtypeStruct((2 * N_PAD, FH), jnp.float32),
        mesh=_mesh(),
        scratch_types=[
            pltpu.VMEM((NBUF, 3, C), jnp.int32),
            pltpu.VMEM((NBUF, C, FH), jnp.float32),
            pltpu.VMEM((C, FH), jnp.float32),
            pltpu.VMEM_SHARED((N_PAD, FH), jnp.float32),
            pltpu.SemaphoreType.DMA((NBUF,)),
            pltpu.SemaphoreType.DMA((NBUF,)),
            pltpu.SemaphoreType.DMA((NBUF,)),
        ],
        compiler_params=_SC_PARAMS,
    )
    def run(edge_ref, h_ref, agg_ref, ebuf, rows, zrows, acc_sh,
            semi, semg, semr):
        c = lax.axis_index("c")
        s = lax.axis_index("s")

        _zero_rows(zrows)
        for t in range(EMB_CHUNKS):
            pltpu.sync_copy(zrows, acc_sh.at[pl.ds(s * ROWS_PER_TILE + t * C, C)])

        plsc.subcore_barrier()

        _edge_phase(edge_ref, h_ref, acc_sh, ebuf, rows, semi, semg, semr, c, s)

        plsc.subcore_barrier()

        _acc_writeout(acc_sh, agg_ref, rows, c, s)

    return run(edge3, h_cat)


BT = 2048
NBLK = N_PAD // BT


def _lospec(w=FH):
    return pl.BlockSpec((BT, w), lambda *g: (g[-1], 0))


def _hispec():
    return pl.BlockSpec((BT, FH), lambda *g: (NBLK + g[-1], 0))


def _fullspec(shape):
    return pl.BlockSpec(shape, lambda *g: (0,) * len(shape))


def _dense1_body(agglo_ref, agghi_ref, hlo_ref, hhi_ref, cnt_ref,
                 wl_lo_ref, wl_hi_ref, wr_lo_ref, wr_hi_ref, b_ref, out_ref):
    half = pl.program_id(0)
    inv = 1.0 / jnp.maximum(cnt_ref[...], 1.0)
    alo = agglo_ref[...] * inv
    ahi = agghi_ref[...] * inv
    out = (jnp.dot(alo, wl_lo_ref[...], preferred_element_type=jnp.float32)
           + jnp.dot(ahi, wl_hi_ref[...], preferred_element_type=jnp.float32)
           + jnp.dot(hlo_ref[...], wr_lo_ref[...], preferred_element_type=jnp.float32)
           + jnp.dot(hhi_ref[...], wr_hi_ref[...], preferred_element_type=jnp.float32)
           + b_ref[...])
    out = jnp.maximum(out, 0.0)
    out_ref[...] = jnp.where(half == 0, out[:, :FH], out[:, FH:])


def _dense1(agg_cat, h_cat, cnt, W_l, W_r, b):
    return pl.pallas_call(
        _dense1_body,
        grid=(2, NBLK),
        in_specs=[_lospec(), _hispec(), _lospec(), _hispec(), _lospec(1),
                  _fullspec((FH, F)), _fullspec((FH, F)),
                  _fullspec((FH, F)), _fullspec((FH, F)), _fullspec((1, F))],
        out_specs=pl.BlockSpec((BT, FH), lambda h, i: (h * NBLK + i, 0)),
        out_shape=jax.ShapeDtypeStruct((2 * N_PAD, FH), jnp.float32),
    )(agg_cat, agg_cat, h_cat, h_cat, cnt,
      W_l[:FH], W_l[FH:], W_r[:FH], W_r[FH:], b.reshape(1, F))


def _dense2_body(agglo_ref, agghi_ref, hlo_ref, hhi_ref, cnt_ref,
                 batch_ref,
                 wl_lo_ref, wl_hi_ref, wr_lo_ref, wr_hi_ref, b_ref,
                 wlin_ref, blin_ref, out_ref, pooled_acc, cntg_acc):
    i = pl.program_id(0)

    @pl.when(i == 0)
    def _():
        pooled_acc[...] = jnp.zeros((G, F), jnp.float32)
        cntg_acc[...] = jnp.zeros((G, 1), jnp.float32)

    inv = 1.0 / jnp.maximum(cnt_ref[...], 1.0)
    alo = agglo_ref[...] * inv
    ahi = agghi_ref[...] * inv
    h2 = (jnp.dot(alo, wl_lo_ref[...], preferred_element_type=jnp.float32)
          + jnp.dot(ahi, wl_hi_ref[...], preferred_element_type=jnp.float32)
          + jnp.dot(hlo_ref[...], wr_lo_ref[...], preferred_element_type=jnp.float32)
          + jnp.dot(hhi_ref[...], wr_hi_ref[...], preferred_element_type=jnp.float32)
          + b_ref[...])
    h2 = jnp.maximum(h2, 0.0)

    bvals = batch_ref[...]
    gids = lax.broadcasted_iota(jnp.int32, (G, BT), 0)
    onehot = (bvals == gids).astype(jnp.float32)
    pooled_acc[...] += jnp.dot(onehot, h2, preferred_element_type=jnp.float32)
    cntg_acc[...] += jnp.sum(onehot, axis=1, keepdims=True)

    @pl.when(i == NBLK - 1)
    def _():
        pooled = pooled_acc[...] / jnp.maximum(cntg_acc[...], 1.0)
        out_ref[...] = (jnp.dot(pooled, wlin_ref[...],
                                preferred_element_type=jnp.float32)
                        + blin_ref[...])


def _dense2(agg_cat, h_cat, cnt, batch_row, W_l, W_r, b, W_lin, b_lin):
    return pl.pallas_call(
        _dense2_body,
        grid=(NBLK,),
        in_specs=[_lospec(), _hispec(), _lospec(), _hispec(), _lospec(1),
                  pl.BlockSpec((1, BT), lambda i: (0, i)),
                  _fullspec((FH, F)), _fullspec((FH, F)),
                  _fullspec((FH, F)), _fullspec((FH, F)), _fullspec((1, F)),
                  _fullspec((F, CLS)), _fullspec((1, CLS))],
        out_specs=pl.BlockSpec((G, CLS), lambda i: (0, 0)),
        out_shape=jax.ShapeDtypeStruct((G, CLS), jnp.float32),
        scratch_shapes=[pltpu.VMEM((G, F), jnp.float32),
                        pltpu.VMEM((G, 1), jnp.float32)],
    )(agg_cat, agg_cat, h_cat, h_cat, cnt, batch_row,
      W_l[:FH], W_l[FH:], W_r[:FH], W_r[FH:], b.reshape(1, F),
      W_lin, b_lin.reshape(1, CLS))


def kernel(x, edge_index, batch, emb_table, W_l1, W_r1, b1, W_l2, W_r2, b2,
           W_lin, b_lin):
    x_pad = jnp.pad(x.astype(jnp.int32), (0, N_PAD - N))
    x2 = jnp.stack([x_pad, x_pad + V])
    edge = edge_index.astype(jnp.int32)
    edge3 = jnp.stack([edge[0], edge[0] + N_PAD, edge[1]])
    emb_cat = jnp.concatenate([emb_table[:, :FH], emb_table[:, FH:]], axis=0)
    batch_row = jnp.pad(batch.astype(jnp.int32), (0, N_PAD - N),
                        constant_values=G).reshape(1, N_PAD)

    h0_cat, agg_cat, cnt = _stage_ab1(x2, edge3, emb_cat)
    cnt = cnt.reshape(N_PAD, 1)

    h1_cat = _dense1(agg_cat, h0_cat, cnt, W_l1, W_r1, b1)

    agg2_cat = _stage_b(edge3, h1_cat)
    out = _dense2(agg2_cat, h1_cat, cnt, batch_row,
                  W_l2, W_r2, b2, W_lin, b_lin)
    return out


# speedup vs baseline: 1.3178x; 1.3178x over previous
import functools

import jax
import jax.numpy as jnp
from jax import lax
from jax.experimental import pallas as pl
from jax.experimental.pallas import tpu as pltpu
from jax.experimental.pallas import tpu_sc as plsc

N = 50000
E = 800000
V = 10000
F = 64
FH = 32
CLS = 10
G = 512

NC = 2
NS = 16
NW = NC * NS

N_PAD = 51200
ROWS_PER_TILE = N_PAD // NS
C = 128
NCHUNK = E // C
EMB_CHUNKS = ROWS_PER_TILE // C
NBUF = 4


def _mesh():
    return plsc.VectorSubcoreMesh(
        core_axis_name="c", subcore_axis_name="s", num_cores=NC, num_subcores=NS)


_SC_PARAMS = pltpu.CompilerParams(use_tc_tiling_on_sc=False)


def _b(j):
    return jnp.bitwise_and(j, NBUF - 1)


def _edge_phase(edge_ref, h_ref, acc_sh, ebuf, rows, semi, semg, semr,
                c, s, ones=None, cnt_sh=None, semo=None):
    lo = s * NCHUNK // NS
    hi = (s + 1) * NCHUNK // NS
    with_ones = ones is not None

    def idx_copy(j):
        return pltpu.make_async_copy(
            edge_ref.at[:, pl.ds(j * C, C)], ebuf.at[_b(j)], semi.at[_b(j)])

    def gather_copy(j):
        return pltpu.make_async_copy(
            h_ref.at[ebuf.at[_b(j), c]], rows.at[_b(j)], semg.at[_b(j)])

    def scat_start(j):
        pltpu.async_copy(
            rows.at[_b(j)], acc_sh.at[ebuf.at[_b(j), 2]], semr.at[_b(j)],
            add=True)

    def scat_wait(j):
        pltpu.make_async_copy(
            rows.at[_b(j)], acc_sh.at[ebuf.at[_b(j), 2]], semr.at[_b(j)]).wait()

    def ones_start(j):
        pltpu.async_copy(
            ones, cnt_sh.at[ebuf.at[_b(j), 2]], semo.at[_b(j)], add=True)

    def ones_wait(j):
        pltpu.make_async_copy(
            ones, cnt_sh.at[ebuf.at[_b(j), 2]], semo.at[_b(j)]).wait()

    @pl.when(lo < hi)
    def _():
        idx_copy(lo).start()

    @pl.when(lo + 1 < hi)
    def _():
        idx_copy(lo + 1).start()

    @pl.when(lo < hi)
    def _():
        idx_copy(lo).wait()
        gather_copy(lo).start()

    def body(j, _):
        @pl.when(j - 2 >= lo)
        def _():
            scat_wait(j - 2)
            if with_ones:
                @pl.when(c == 0)
                def _():
                    ones_wait(j - 2)

        @pl.when(j + 2 < hi)
        def _():
            idx_copy(j + 2).start()

        @pl.when(j + 1 < hi)
        def _():
            idx_copy(j + 1).wait()
            gather_copy(j + 1).start()

        gather_copy(j).wait()
        scat_start(j)
        if with_ones:
            @pl.when(c == 0)
            def _():
                ones_start(j)
        return 0
    lax.fori_loop(lo, hi, body, 0)

    def drain(j, _):
        scat_wait(j)
        if with_ones:
            @pl.when(c == 0)
            def _():
                ones_wait(j)
        return 0
    lax.fori_loop(jnp.maximum(lo, hi - 2), hi, drain, 0)


def _zero_rows(zrows):
    def zloop(r, _):
        zrows[r, pl.ds(0, 16)] = jnp.zeros((16,), jnp.float32)
        zrows[r, pl.ds(16, 16)] = jnp.zeros((16,), jnp.float32)
        return 0
    lax.fori_loop(0, C, zloop, 0)


def _acc_writeout(acc_sh, out_ref, rows, c, s):
    for t in range(EMB_CHUNKS):
        base = s * ROWS_PER_TILE + t * C
        pltpu.sync_copy(acc_sh.at[pl.ds(base, C)], rows.at[0])
        pltpu.sync_copy(rows.at[0], out_ref.at[pl.ds(c * N_PAD + base, C)])


def _stage_ab1(x2, edge3, emb_cat):
    @functools.partial(
        pl.kernel,
        out_type=(
            jax.ShapeDtypeStruct((2 * N_PAD, FH), jnp.float32),
            jax.ShapeDtypeStruct((2 * N_PAD, FH), jnp.float32),
            jax.ShapeDtypeStruct((N_PAD,), jnp.float32),
        ),
        mesh=_mesh(),
        scratch_types=[
            pltpu.VMEM((NBUF, 2, C), jnp.int32),
            pltpu.VMEM((NBUF, 3, C), jnp.int32),
            pltpu.VMEM((NBUF, C, FH), jnp.float32),
            pltpu.VMEM((C, FH), jnp.float32),
            pltpu.VMEM((C,), jnp.float32),
            pltpu.VMEM((800,), jnp.float32),
            pltpu.VMEM((800,), jnp.float32),
            pltpu.VMEM_SHARED((N_PAD, FH), jnp.float32),
            pltpu.VMEM_SHARED((N_PAD,), jnp.float32),
            pltpu.SemaphoreType.DMA((NBUF,)),
            pltpu.SemaphoreType.DMA((NBUF,)),
            pltpu.SemaphoreType.DMA((NBUF,)),
            pltpu.SemaphoreType.DMA((NBUF,)),
            pltpu.SemaphoreType.DMA((NBUF,)),
        ],
        compiler_params=_SC_PARAMS,
    )
    def run(x_ref, edge_ref, emb_ref, h0_ref, agg_ref, cnt_ref,
            xbuf, ebuf, rows, zrows, ones, zbuf, sbuf, acc_sh, cnt_sh,
            semx, semi, semg, semr, semo):
        c = lax.axis_index("c")
        s = lax.axis_index("s")

        _zero_rows(zrows)

        def oloop(i, _):
            ones[pl.ds(i * 16, 16)] = jnp.ones((16,), jnp.float32)
            return 0
        lax.fori_loop(0, C // 16, oloop, 0)

        def zvloop(i, _):
            zbuf[pl.ds(i * 16, 16)] = jnp.zeros((16,), jnp.float32)
            return 0
        lax.fori_loop(0, 50, zvloop, 0)

        for t in range(EMB_CHUNKS):
            pltpu.sync_copy(zrows, acc_sh.at[pl.ds(s * ROWS_PER_TILE + t * C, C)])
        for t in range(4):
            pltpu.sync_copy(zbuf, cnt_sh.at[pl.ds(s * ROWS_PER_TILE + t * 800, 800)])

        def xload(k):
            base = s * ROWS_PER_TILE + k * C
            return pltpu.make_async_copy(
                x_ref.at[:, pl.ds(base, C)], xbuf.at[_b(k)], semx.at[_b(k)])

        def embg(k):
            return pltpu.make_async_copy(
                emb_ref.at[xbuf.at[_b(k), c]], rows.at[_b(k)], semg.at[_b(k)])

        xload(0).start()
        xload(1).start()
        xload(0).wait()
        embg(0).start()

        def emb_body(k, _):
            @pl.when(k + 2 < EMB_CHUNKS)
            def _():
                xload(k + 2).start()

            @pl.when(k + 1 < EMB_CHUNKS)
            def _():
                xload(k + 1).wait()
                embg(k + 1).start()

            embg(k).wait()
            base = s * ROWS_PER_TILE + k * C
            pltpu.sync_copy(rows.at[_b(k)],
                            h0_ref.at[pl.ds(c * N_PAD + base, C)])
            return 0
        lax.fori_loop(0, EMB_CHUNKS, emb_body, 0)

        plsc.subcore_barrier()

        _edge_phase(edge_ref, h0_ref, acc_sh, ebuf, rows, semi, semg, semr,
                    c, s, ones=ones, cnt_sh=cnt_sh, semo=semo)

        plsc.subcore_barrier()

        _acc_writeout(acc_sh, agg_ref, rows, c, s)

        @pl.when(c == 0)
        def _():
            for t in range(4):
                sl = pl.ds(s * ROWS_PER_TILE + t * 800, 800)
                pltpu.sync_copy(cnt_sh.at[sl], sbuf)
                pltpu.sync_copy(sbuf, cnt_ref.at[sl])

    return run(x2, edge3, emb_cat)


def _stage_b(edge3, h_cat):
    @functools.partial(
        pl.kernel,
        out_type=jax.ShapeDtypeStruct((2 * N_PAD, FH), jnp.float32),
        mesh=_mesh(),
        scratch_types=[
            pltpu.VMEM((NBUF, 3, C), jnp.int32),
            pltpu.VMEM((NBUF, C, FH), jnp.float32),
            pltpu.VMEM((C, FH), jnp.float32),
            pltpu.VMEM_SHARED((N_PAD, FH), jnp.float32),
            pltpu.SemaphoreType.DMA((NBUF,)),
            pltpu.SemaphoreType.DMA((NBUF,)),
            pltpu.SemaphoreType.DMA((NBUF,)),
        ],
        compiler_params=_SC_PARAMS,
    )
    def run(edge_ref, h_ref, agg_ref, ebuf, rows, zrows, acc_sh,
            semi, semg, semr):
        c = lax.axis_index("c")
        s = lax.axis_index("s")

        _zero_rows(zrows)
        for t in range(EMB_CHUNKS):
            pltpu.sync_copy(zrows, acc_sh.at[pl.ds(s * ROWS_PER_TILE + t * C, C)])

        plsc.subcore_barrier()

        _edge_phase(edge_ref, h_ref, acc_sh, ebuf, rows, semi, semg, semr, c, s)

        plsc.subcore_barrier()

        _acc_writeout(acc_sh, agg_ref, rows, c, s)

    return run(edge3, h_cat)


BT = 2048
NBLK = N_PAD // BT
BT4 = BT // 4
MROWS = N_PAD * FH // 128


def _lospec():
    return pl.BlockSpec((BT4, 128), lambda *g: (g[-1], 0))


def _hispec():
    return pl.BlockSpec((BT4, 128), lambda *g: (MROWS // BT4 + g[-1], 0))


def _cntspec():
    return pl.BlockSpec((BT4, 4), lambda *g: (g[-1], 0))


def _fullspec(shape):
    return pl.BlockSpec(shape, lambda *g: (0,) * len(shape))


def _dense1_body(agglo_ref, agghi_ref, hlo_ref, hhi_ref, cnt_ref, rep_ref,
                 wla_ref, wlb_ref, wra_ref, wrb_ref, b_ref, out_ref):
    half = pl.program_id(0)
    bias = jnp.where(half == 0, b_ref[0:1], b_ref[1:2])
    inv4 = 1.0 / jnp.maximum(cnt_ref[...], 1.0)
    invt = jnp.dot(inv4, rep_ref[...],
                   preferred_element_type=jnp.float32)
    alo = agglo_ref[...] * invt
    ahi = agghi_ref[...] * invt
    out = (jnp.dot(alo, wla_ref[0], preferred_element_type=jnp.float32)
           + jnp.dot(ahi, wlb_ref[0], preferred_element_type=jnp.float32)
           + jnp.dot(hlo_ref[...], wra_ref[0], preferred_element_type=jnp.float32)
           + jnp.dot(hhi_ref[...], wrb_ref[0], preferred_element_type=jnp.float32)
           + bias)
    out_ref[...] = jnp.maximum(out, 0.0)


def _dense1(agg4, h04, cnt4, rep, wla, wlb, wra, wrb, bt):
    wspec = pl.BlockSpec((1, 128, 128), lambda h, i: (h, 0, 0))
    return pl.pallas_call(
        _dense1_body,
        grid=(2, NBLK),
        in_specs=[_lospec(), _hispec(), _lospec(), _hispec(), _cntspec(),
                  _fullspec((4, 128)),
                  wspec, wspec, wspec, wspec,
                  _fullspec((2, 128))],
        out_specs=pl.BlockSpec((BT4, 128), lambda h, i: (h * NBLK + i, 0)),
        out_shape=jax.ShapeDtypeStruct((2 * MROWS, 128), jnp.float32),
    )(agg4, agg4, h04, h04, cnt4, rep, wla, wlb, wra, wrb, bt)


def _dense2_body(agglo_ref, agghi_ref, hlo_ref, hhi_ref, cnt_ref, rep_ref,
                 batch_ref,
                 wla_ref, wlb_ref, wra_ref, wrb_ref, b_ref,
                 wlin_ref, blin_ref, out_ref, pooled_acc, cntg_acc):
    i = pl.program_id(0)

    @pl.when(i == 0)
    def _():
        pooled_acc[...] = jnp.zeros((G, F), jnp.float32)
        cntg_acc[...] = jnp.zeros((G, 1), jnp.float32)

    inv4 = 1.0 / jnp.maximum(cnt_ref[...], 1.0)
    invt = jnp.dot(inv4, rep_ref[...], preferred_element_type=jnp.float32)
    alo = agglo_ref[...] * invt
    ahi = agghi_ref[...] * invt
    h2lo = (jnp.dot(alo, wla_ref[0], preferred_element_type=jnp.float32)
            + jnp.dot(ahi, wlb_ref[0], preferred_element_type=jnp.float32)
            + jnp.dot(hlo_ref[...], wra_ref[0], preferred_element_type=jnp.float32)
            + jnp.dot(hhi_ref[...], wrb_ref[0], preferred_element_type=jnp.float32)
            + b_ref[0:1])
    h2lo = jnp.maximum(h2lo, 0.0)
    h2hi = (jnp.dot(alo, wla_ref[1], preferred_element_type=jnp.float32)
            + jnp.dot(ahi, wlb_ref[1], preferred_element_type=jnp.float32)
            + jnp.dot(hlo_ref[...], wra_ref[1], preferred_element_type=jnp.float32)
            + jnp.dot(hhi_ref[...], wrb_ref[1], preferred_element_type=jnp.float32)
            + b_ref[1:2])
    h2hi = jnp.maximum(h2hi, 0.0)

    bk = batch_ref[...]
    gids = lax.broadcasted_iota(jnp.int32, (G, BT4), 0)
    plo = jnp.zeros((G, FH), jnp.float32)
    phi = jnp.zeros((G, FH), jnp.float32)
    cg = jnp.zeros((G, 1), jnp.float32)
    for k in range(4):
        ohk = (bk[k:k + 1, :] == gids).astype(jnp.float32)
        plo += jnp.dot(ohk, h2lo[:, FH * k:FH * (k + 1)],
                       preferred_element_type=jnp.float32)
        phi += jnp.dot(ohk, h2hi[:, FH * k:FH * (k + 1)],
                       preferred_element_type=jnp.float32)
        cg += jnp.sum(ohk, axis=1, keepdims=True)
    pooled_acc[...] += jnp.concatenate([plo, phi], axis=1)
    cntg_acc[...] += cg

    @pl.when(i == NBLK - 1)
    def _():
        pooled = pooled_acc[...] / jnp.maximum(cntg_acc[...], 1.0)
        out_ref[...] = (jnp.dot(pooled, wlin_ref[...],
                                preferred_element_type=jnp.float32)
                        + blin_ref[...])


def _dense2(agg4, h14, cnt4, rep, batch_t, wla, wlb, wra, wrb, bt,
            W_lin, b_lin):
    return pl.pallas_call(
        _dense2_body,
        grid=(NBLK,),
        in_specs=[_lospec(), _hispec(), _lospec(), _hispec(), _cntspec(),
                  _fullspec((4, 128)),
                  pl.BlockSpec((4, BT4), lambda i: (0, i)),
                  _fullspec((2, 128, 128)), _fullspec((2, 128, 128)),
                  _fullspec((2, 128, 128)), _fullspec((2, 128, 128)),
                  _fullspec((2, 128)),
                  _fullspec((F, CLS)), _fullspec((1, CLS))],
        out_specs=pl.BlockSpec((G, CLS), lambda i: (0, 0)),
        out_shape=jax.ShapeDtypeStruct((G, CLS), jnp.float32),
        scratch_shapes=[pltpu.VMEM((G, F), jnp.float32),
                        pltpu.VMEM((G, 1), jnp.float32)],
    )(agg4, agg4, h14, h14, cnt4, rep, batch_t,
      wla, wlb, wra, wrb, bt, W_lin, b_lin.reshape(1, CLS))


def _bd_weights(W, b):
    eye4 = jnp.eye(4, dtype=jnp.float32)
    wla = jnp.stack([jnp.kron(eye4, W[:FH, :FH]), jnp.kron(eye4, W[:FH, FH:])])
    wlb = jnp.stack([jnp.kron(eye4, W[FH:, :FH]), jnp.kron(eye4, W[FH:, FH:])])
    bt = jnp.stack([jnp.tile(b[:FH], 4), jnp.tile(b[FH:], 4)])
    return wla, wlb, bt


def kernel(x, edge_index, batch, emb_table, W_l1, W_r1, b1, W_l2, W_r2, b2,
           W_lin, b_lin):
    x_pad = jnp.pad(x.astype(jnp.int32), (0, N_PAD - N))
    x2 = jnp.stack([x_pad, x_pad + V])
    edge = edge_index.astype(jnp.int32)
    edge3 = jnp.stack([edge[0], edge[0] + N_PAD, edge[1]])
    emb_cat = jnp.concatenate([emb_table[:, :FH], emb_table[:, FH:]], axis=0)
    batch_pad = jnp.pad(batch.astype(jnp.int32), (0, N_PAD - N),
                        constant_values=G)
    batch_t = batch_pad.reshape(N_PAD // 4, 4).T
    rep = jnp.kron(jnp.eye(4, dtype=jnp.float32), jnp.ones((1, FH), jnp.float32))

    wla1, wlb1, bt1 = _bd_weights(W_l1, b1)
    wra1, wrb1, _ = _bd_weights(W_r1, b1)
    wla2, wlb2, bt2 = _bd_weights(W_l2, b2)
    wra2, wrb2, _ = _bd_weights(W_r2, b2)

    h0_cat, agg_cat, cnt = _stage_ab1(x2, edge3, emb_cat)
    cnt4 = cnt.reshape(N_PAD // 4, 4)
    agg4 = agg_cat.reshape(2 * MROWS, 128)
    h04 = h0_cat.reshape(2 * MROWS, 128)

    h14 = _dense1(agg4, h04, cnt4, rep, wla1, wlb1, wra1, wrb1, bt1)
    h1_cat = h14.reshape(2 * N_PAD, FH)

    agg2_cat = _stage_b(edge3, h1_cat)
    agg24 = agg2_cat.reshape(2 * MROWS, 128)
    out = _dense2(agg24, h14, cnt4, rep, batch_t,
                  wla2, wlb2, wra2, wrb2, bt2, W_lin, b_lin)
    return out
